# Initial kernel scaffold; baseline (speedup 1.0000x reference)
import jax
import jax.numpy as jnp
from jax.experimental import pallas as pl


def kernel(features, proj_w, avg_features):
    raise NotImplementedError("write your pallas kernel here")


# speedup vs baseline: 3.9242x; 3.9242x over previous
import functools

import jax
import jax.numpy as jnp
from jax import lax
from jax.experimental import pallas as pl
from jax.experimental.pallas import tpu as pltpu

_T = 4096
_D = 768
_K = 512
_C = 20
_BLK = 512
_NBLK = _T // _BLK
_KS = 9
_PAD = _KS // 2
_MOM = 0.95
_CONF_TH = 0.7


def _stage1_body(feat_ref, w_ref, avg_ref, proj_ref, sums_ref, cnt_ref, cov_ref):
    i = pl.program_id(0)
    x = feat_ref[...]
    f = x / jnp.maximum(jnp.sqrt(jnp.sum(x * x, axis=1, keepdims=True)), 1e-12)
    p = lax.dot_general(f, w_ref[...], (((1,), (1,)), ((), ())),
                        preferred_element_type=jnp.float32)
    pn = p / jnp.maximum(jnp.sqrt(jnp.sum(p * p, axis=1, keepdims=True)), 1e-12)
    proj_ref[...] = pn

    sims = lax.dot_general(pn, avg_ref[...], (((1,), (1,)), ((), ())),
                           preferred_element_type=jnp.float32)
    m = jnp.max(sims, axis=1, keepdims=True)
    e = jnp.exp((sims - m) / 0.1)
    s = jnp.sum(e, axis=1, keepdims=True)
    conf = e / s
    max_conf = jnp.max(conf, axis=1)
    preds = jnp.argmax(conf, axis=1)
    mask = max_conf > _CONF_TH

    classes = lax.broadcasted_iota(jnp.int32, (_BLK, _C), 1)
    onehot = jnp.where((preds[:, None] == classes) & mask[:, None], 1.0, 0.0)

    @pl.when(i == 0)
    def _init():
        sums_ref[...] = jnp.zeros_like(sums_ref)
        cnt_ref[...] = jnp.zeros_like(cnt_ref)
        cov_ref[...] = jnp.zeros_like(cov_ref)

    sums_ref[...] += lax.dot_general(onehot, pn, (((0,), (0,)), ((), ())),
                                     preferred_element_type=jnp.float32)
    ones = jnp.ones((_BLK, 128), jnp.float32)
    cnt_ref[...] += lax.dot_general(onehot, ones, (((0,), (0,)), ((), ())),
                                    preferred_element_type=jnp.float32)
    cov_ref[...] += lax.dot_general(f[:, :_K], f, (((0,), (0,)), ((), ())),
                                    preferred_element_type=jnp.float32)


def _stage2_body(proj_ref, sums_ref, cnt_ref, avg_ref, w_ref, cov_ref,
                 sm_ref, ids_ref, newproj_ref, pad_ref):
    sums = sums_ref[...]
    counts_col = cnt_ref[:, 0:1]
    avg = avg_ref[...]

    means = sums / jnp.maximum(counts_col, 1.0)
    upd = _MOM * avg + (1.0 - _MOM) * means
    updn = upd / jnp.maximum(jnp.sqrt(jnp.sum(upd * upd, axis=1, keepdims=True)), 1e-12)
    new_avg = jnp.where(counts_col > 0.0, updn, avg)

    simil = lax.dot_general(proj_ref[...], new_avg, (((1,), (1,)), ((), ())),
                            preferred_element_type=jnp.float32)
    class_sims = jnp.max(simil, axis=1)
    ids_ref[...] = jnp.argmax(simil, axis=1).astype(jnp.int32).reshape(1, _T)

    pad_ref[0, pl.ds(_PAD, _T)] = class_sims
    for k in range(_PAD):
        pad_ref[0, k:k + 1] = pad_ref[0, 2 * _PAD - k:2 * _PAD - k + 1]
        pad_ref[0, _PAD + _T + k:_PAD + _T + k + 1] = \
            pad_ref[0, _PAD + _T - 2 - k:_PAD + _T - 1 - k]
    acc = pad_ref[0, pl.ds(0, _T)]
    for j in range(1, _KS):
        acc = acc + pad_ref[0, pl.ds(j, _T)]
    sm_ref[...] = (acc * (1.0 / _KS)).reshape(1, _T)

    any_mask = jnp.sum(counts_col) > 0.0
    newproj_ref[...] = jnp.where(any_mask,
                                 _MOM * w_ref[...] + (1.0 - _MOM) * cov_ref[...],
                                 w_ref[...])


@functools.partial(jax.jit, static_argnames=("interpret",))
def kernel(features, proj_w, avg_features, interpret=False):
    proj, sums, counts, cov = pl.pallas_call(
        _stage1_body,
        grid=(_NBLK,),
        in_specs=[
            pl.BlockSpec((_BLK, _D), lambda i: (i, 0)),
            pl.BlockSpec((_K, _D), lambda i: (0, 0)),
            pl.BlockSpec((_C, _K), lambda i: (0, 0)),
        ],
        out_specs=[
            pl.BlockSpec((_BLK, _K), lambda i: (i, 0)),
            pl.BlockSpec((_C, _K), lambda i: (0, 0)),
            pl.BlockSpec((_C, 128), lambda i: (0, 0)),
            pl.BlockSpec((_K, _D), lambda i: (0, 0)),
        ],
        out_shape=[
            jax.ShapeDtypeStruct((_T, _K), jnp.float32),
            jax.ShapeDtypeStruct((_C, _K), jnp.float32),
            jax.ShapeDtypeStruct((_C, 128), jnp.float32),
            jax.ShapeDtypeStruct((_K, _D), jnp.float32),
        ],
        interpret=interpret,
    )(features, proj_w, avg_features)

    smoothed, class_ids, new_proj = pl.pallas_call(
        _stage2_body,
        out_shape=[
            jax.ShapeDtypeStruct((1, _T), jnp.float32),
            jax.ShapeDtypeStruct((1, _T), jnp.int32),
            jax.ShapeDtypeStruct((_K, _D), jnp.float32),
        ],
        scratch_shapes=[pltpu.VMEM((1, _T + 2 * _PAD), jnp.float32)],
        interpret=interpret,
    )(proj, sums, counts, avg_features, proj_w, cov)

    return smoothed.reshape(_T), class_ids.reshape(_T), new_proj
